# baseline (speedup 1.0000x reference)
import jax
import jax.numpy as jnp
from jax.experimental import pallas as pl


def _tc_body(x_ref, w_ref, o_ref):
    xb = x_ref[...]
    s = jnp.sum(xb, axis=2)
    o_ref[...] = jax.lax.dot_general(
        s, w_ref[...],
        dimension_numbers=(((1,), (1,)), ((), ())),
        preferred_element_type=jnp.float32,
    ) * (1.0 / x_ref.shape[2])


def kernel(x, W):
    B, C, H, Wsp = x.shape
    S = H * Wsp
    E = W.shape[0]
    xr = x.reshape(B, C, S)
    Bblk = 8
    return pl.pallas_call(
        _tc_body,
        grid=(B // Bblk,),
        in_specs=[
            pl.BlockSpec((Bblk, C, S), lambda i: (i, 0, 0)),
            pl.BlockSpec((E, C), lambda i: (0, 0)),
        ],
        out_specs=pl.BlockSpec((Bblk, E), lambda i: (i, 0)),
        out_shape=jax.ShapeDtypeStruct((B, E), jnp.float32),
    )(xr, W)
